# baseline (speedup 1.0000x reference)
import functools

import jax
import jax.numpy as jnp
from jax import lax
from jax.experimental import pallas as pl
from jax.experimental.pallas import tpu as pltpu
from jax.experimental.pallas import tpu_sc as plsc

BATCH = 4096
SEQ = 50
EMBED = 64
VOCAB = 1000000
PADW = 128
LANES = 16
NGROUP = EMBED // LANES

NC, NS = 2, 16
NW = NC * NS
ROWS_W = BATCH // NW
CB = 2
CHUNK_TOK = CB * SEQ
LIST_LEN = 112
NCHUNK = ROWS_W // CB
TOK_W = ROWS_W * SEQ
NBUF = 3
SLAB0 = 500224


def _make_body(lo, hi, final):
    nrow = hi - lo

    def _body(idx_hbm, mask_hbm, table_hbm, *rest):
        if final:
            partial_hbm, out_hbm = rest[0], rest[1]
            scratch = rest[2:]
        else:
            out_hbm = rest[0]
            scratch = rest[1:]
        idx_v, mask_v = scratch[0], scratch[1]
        lists = scratch[2:2 + NBUF]
        rows_b = scratch[2 + NBUF:2 + 2 * NBUF]
        out_v = scratch[2 + 2 * NBUF]
        sems = scratch[3 + 2 * NBUF:3 + 3 * NBUF]

        wid = lax.axis_index("s") * NC + lax.axis_index("c")

        pltpu.sync_copy(idx_hbm.at[pl.ds(wid * TOK_W, TOK_W)],
                        idx_v.at[pl.ds(0, TOK_W)])
        pltpu.sync_copy(mask_hbm.at[pl.ds(wid * TOK_W, TOK_W)],
                        mask_v.at[pl.ds(0, TOK_W)])
        if final:
            pltpu.sync_copy(partial_hbm.at[pl.ds(wid * ROWS_W, ROWS_W), :],
                            out_v)

        def start(g, b):
            for t in range(LIST_LEN // LANES):
                iv = idx_v[pl.ds(g * CHUNK_TOK + t * LANES, LANES)]
                lists[b][pl.ds(t * LANES, LANES)] = jnp.clip(
                    iv - lo, 0, nrow - 1)
            pltpu.make_async_copy(
                table_hbm.at[lists[b]], rows_b[b], sems[b]).start()

        def compute(g, b):
            rows = rows_b[b]
            zero = jnp.zeros((LANES,), jnp.float32)

            def s_step(s, acc_all):
                new = []
                for j in range(CB):
                    a = acc_all[j]
                    tok = g * CHUNK_TOK + j * SEQ + s
                    m = mask_v[pl.ds(tok, LANES)][0]
                    iv = idx_v[pl.ds(tok, LANES)][0]
                    inslab = (iv >= lo) & (iv < hi)
                    me = jnp.where(inslab, m, 0.0)
                    r = j * SEQ + s
                    vals = [a[k] + rows[r, pl.ds(k * LANES, LANES)] * me
                            for k in range(NGROUP)]
                    vals.append(a[NGROUP] + m)
                    new.append(tuple(vals))
                return tuple(new)

            init = tuple(
                tuple(zero for _ in range(NGROUP)) + (jnp.float32(0.0),)
                for _ in range(CB))
            acc_all = lax.fori_loop(0, SEQ, s_step, init)
            for j in range(CB):
                row_o = g * CB + j
                if final:
                    denom = jnp.broadcast_to(
                        jnp.maximum(acc_all[j][NGROUP], 1.0), (LANES,))
                    for k in range(NGROUP):
                        sl = pl.ds(k * LANES, LANES)
                        out_v[row_o, sl] = (out_v[row_o, sl]
                                            + acc_all[j][k]) / denom
                else:
                    for k in range(NGROUP):
                        sl = pl.ds(k * LANES, LANES)
                        out_v[row_o, sl] = acc_all[j][k]

        for b in range(NBUF):
            start(b, b)

        def tb(t, carry):
            for b in range(NBUF):
                g = NBUF * t + b
                pltpu.make_async_copy(
                    table_hbm.at[lists[b]], rows_b[b], sems[b]).wait()
                compute(g, b)

                @pl.when(g + NBUF < NCHUNK)
                def _():
                    start(g + NBUF, b)
            return carry

        lax.fori_loop(0, NCHUNK // NBUF, tb, 0)

        g_last = NCHUNK - 1
        b_last = g_last % NBUF
        pltpu.make_async_copy(
            table_hbm.at[lists[b_last]], rows_b[b_last], sems[b_last]).wait()
        compute(g_last, b_last)

        pltpu.sync_copy(out_v, out_hbm.at[pl.ds(wid * ROWS_W, ROWS_W), :])

    return _body


def _make_call(lo, hi, final):
    mesh = plsc.VectorSubcoreMesh(core_axis_name="c", subcore_axis_name="s")
    listbuf = pltpu.VMEM((LIST_LEN,), jnp.int32)
    rowbuf = pltpu.VMEM((LIST_LEN, PADW), jnp.float32)
    return pl.kernel(
        _make_body(lo, hi, final),
        out_type=jax.ShapeDtypeStruct((BATCH, EMBED), jnp.float32),
        mesh=mesh,
        scratch_types=(
            [pltpu.VMEM((TOK_W + LANES,), jnp.int32),
             pltpu.VMEM((TOK_W + LANES,), jnp.float32)]
            + [listbuf] * NBUF + [rowbuf] * NBUF
            + [pltpu.VMEM((ROWS_W, EMBED), jnp.float32)]
            + [pltpu.SemaphoreType.DMA] * NBUF
        ),
        compiler_params=pltpu.CompilerParams(use_tc_tiling_on_sc=False),
    )


@jax.jit
def _embed(idxf, maskf, slab0, slab1):
    p = _make_call(0, SLAB0, False)(idxf, maskf, slab0)
    return _make_call(SLAB0, VOCAB, True)(idxf, maskf, slab1, p)


def kernel(token_indices, mask, embedding_table):
    idxf = token_indices.reshape(-1)
    maskf = mask.reshape(-1)
    pad = ((0, 0), (0, PADW - EMBED))
    slab0 = jnp.pad(embedding_table[:SLAB0], pad)
    slab1 = jnp.pad(embedding_table[SLAB0:], pad)
    return _embed(idxf, maskf, slab0, slab1)


# speedup vs baseline: 15.8838x; 15.8838x over previous
import jax
import jax.numpy as jnp
from jax import lax
from jax.experimental import pallas as pl
from jax.experimental.pallas import tpu as pltpu
from jax.experimental.pallas import tpu_sc as plsc

BATCH = 4096
SEQ = 50
EMBED = 64
PADW = 128
LANES = 16
NGROUP = EMBED // LANES

NC, NS = 2, 16
NW = NC * NS
ROWS_W = BATCH // NW
CB = 2
CHUNK_TOK = CB * SEQ
NCHUNK = ROWS_W // CB
TOK_W = ROWS_W * SEQ
NBUF = 4


def _body(idx_hbm, mask_hbm, table_hbm, out_hbm, idx_v, mask_v, rows0, rows1,
          rows2, rows3, out_v, sem0, sem1, sem2, sem3):
    wid = lax.axis_index("s") * NC + lax.axis_index("c")
    rows_b = (rows0, rows1, rows2, rows3)
    sems = (sem0, sem1, sem2, sem3)

    pltpu.sync_copy(idx_hbm.at[pl.ds(wid * NCHUNK, NCHUNK), :], idx_v)
    pltpu.sync_copy(mask_hbm.at[pl.ds(wid * TOK_W, TOK_W)],
                    mask_v.at[pl.ds(0, TOK_W)])

    def start(g, b):
        pltpu.make_async_copy(
            table_hbm.at[idx_v.at[g]], rows_b[b], sems[b]).start()

    def compute(g, b):
        rows = rows_b[b]
        zero = jnp.zeros((LANES,), jnp.float32)

        def s_step(s, acc_all):
            new = []
            for j in range(CB):
                a = acc_all[j]
                tok = g * CHUNK_TOK + j * SEQ + s
                m = mask_v[pl.ds(tok, LANES)][0]
                r = j * SEQ + s
                vals = [a[k] + rows[r, pl.ds(k * LANES, LANES)] * m
                        for k in range(NGROUP)]
                vals.append(a[NGROUP] + m)
                new.append(tuple(vals))
            return tuple(new)

        init = tuple(tuple(zero for _ in range(NGROUP)) + (jnp.float32(0.0),)
                     for _ in range(CB))
        acc_all = lax.fori_loop(0, SEQ, s_step, init)
        for j in range(CB):
            denom = jnp.broadcast_to(
                jnp.maximum(acc_all[j][NGROUP], 1.0), (LANES,))
            for k in range(NGROUP):
                out_v[g * CB + j, pl.ds(k * LANES, LANES)] = (
                    acc_all[j][k] / denom)

    for b in range(NBUF):
        start(b, b)

    def tb(t, carry):
        for b in range(NBUF):
            g = NBUF * t + b
            pltpu.make_async_copy(
                table_hbm.at[idx_v.at[g]], rows_b[b], sems[b]).wait()
            compute(g, b)

            @pl.when(g + NBUF < NCHUNK)
            def _():
                start(g + NBUF, b)
        return carry

    lax.fori_loop(0, NCHUNK // NBUF, tb, 0)

    pltpu.sync_copy(out_v, out_hbm.at[pl.ds(wid * ROWS_W, ROWS_W), :])


@jax.jit
def _embed(idx2, maskf, table2):
    mesh = plsc.VectorSubcoreMesh(core_axis_name="c", subcore_axis_name="s")
    f = pl.kernel(
        _body,
        out_type=jax.ShapeDtypeStruct((BATCH, EMBED), jnp.float32),
        mesh=mesh,
        scratch_types=[
            pltpu.VMEM((NCHUNK, CHUNK_TOK), jnp.int32),
            pltpu.VMEM((TOK_W + LANES,), jnp.float32),
            pltpu.VMEM((CHUNK_TOK, PADW), jnp.float32),
            pltpu.VMEM((CHUNK_TOK, PADW), jnp.float32),
            pltpu.VMEM((CHUNK_TOK, PADW), jnp.float32),
            pltpu.VMEM((CHUNK_TOK, PADW), jnp.float32),
            pltpu.VMEM((ROWS_W, EMBED), jnp.float32),
            pltpu.SemaphoreType.DMA,
            pltpu.SemaphoreType.DMA,
            pltpu.SemaphoreType.DMA,
            pltpu.SemaphoreType.DMA,
        ],
        compiler_params=pltpu.CompilerParams(use_tc_tiling_on_sc=False),
    )
    return f(idx2, maskf, table2)


def kernel(token_indices, mask, embedding_table):
    idx2 = token_indices.reshape(BATCH // CB, CHUNK_TOK)
    maskf = mask.reshape(-1)
    table2 = jnp.pad(embedding_table, ((0, 0), (0, PADW - EMBED)))
    return _embed(idx2, maskf, table2)
